# Initial kernel scaffold; baseline (speedup 1.0000x reference)
import jax
import jax.numpy as jnp
from jax.experimental import pallas as pl


def kernel(x, edge_index, edge_attr, ew0, eb0, ew1, eb1, ew2, eb2, ew3, eb3, eg, ebt, nw0, nb0, nw1, nb1, nw2, nb2, nw3, nb3, ng, nbt):
    raise NotImplementedError("write your pallas kernel here")


# speedup vs baseline: 1.1424x; 1.1424x over previous
import functools

import jax
import jax.numpy as jnp
from jax.experimental import pallas as pl
from jax.experimental.pallas import tpu as pltpu

EPS = 1e-5


def _edge_mlp_body(a_ref, w0, b0, w1, b1, w2, b2, w3, b3, g, bt, out_ref):
    a = a_ref[...]
    h = jnp.maximum(jnp.dot(a, w0[...], preferred_element_type=jnp.float32) + b0[...], 0.0)
    h = jnp.maximum(jnp.dot(h, w1[...], preferred_element_type=jnp.float32) + b1[...], 0.0)
    h = jnp.maximum(jnp.dot(h, w2[...], preferred_element_type=jnp.float32) + b2[...], 0.0)
    h = jnp.dot(h, w3[...], preferred_element_type=jnp.float32) + b3[...]
    mu = jnp.mean(h, axis=-1, keepdims=True)
    var = jnp.mean((h - mu) ** 2, axis=-1, keepdims=True)
    ln = (h - mu) * jax.lax.rsqrt(var + EPS) * g[...] + bt[...]
    out_ref[...] = a + ln


def _node_mlp_body(x_ref, s0, s1, c0, c1, w0a, w0b, b0, w1, b1, w2, b2, w3, b3,
                   g, bt, out_ref):
    x = x_ref[...]
    cnt = jnp.maximum(c0[...][:, :1] + c1[...][:, :1], 1.0)
    agg = (s0[...] + s1[...]) / cnt
    h = jnp.dot(x, w0a[...], preferred_element_type=jnp.float32)
    h = h + jnp.dot(agg, w0b[...], preferred_element_type=jnp.float32)
    h = jnp.maximum(h + b0[...], 0.0)
    h = jnp.maximum(jnp.dot(h, w1[...], preferred_element_type=jnp.float32) + b1[...], 0.0)
    h = jnp.maximum(jnp.dot(h, w2[...], preferred_element_type=jnp.float32) + b2[...], 0.0)
    h = jnp.dot(h, w3[...], preferred_element_type=jnp.float32) + b3[...]
    mu = jnp.mean(h, axis=-1, keepdims=True)
    var = jnp.mean((h - mu) ** 2, axis=-1, keepdims=True)
    ln = (h - mu) * jax.lax.rsqrt(var + EPS) * g[...] + bt[...]
    out_ref[...] = x + ln


def _full(shape):
    return pl.BlockSpec(shape, lambda i: (0,) * len(shape))


def _edge_mlp(edge_attr, ew, eb, eg, ebt, tile):
    E, H = edge_attr.shape
    grid = E // tile
    specs = [pl.BlockSpec((tile, H), lambda i: (i, 0))]
    args = [edge_attr]
    for w, b in zip(ew, eb):
        specs += [_full((H, H)), _full((1, H))]
        args += [w, b.reshape(1, H)]
    specs += [_full((1, H)), _full((1, H))]
    args += [eg.reshape(1, H), ebt.reshape(1, H)]
    return pl.pallas_call(
        _edge_mlp_body,
        grid=(grid,),
        in_specs=specs,
        out_specs=pl.BlockSpec((tile, H), lambda i: (i, 0)),
        out_shape=jax.ShapeDtypeStruct((E, H), jnp.float32),
    )(*args)


def _node_mlp(x, s0, s1, c0, c1, nw, nb, ng, nbt, tile):
    N, H = x.shape
    grid = N // tile
    row = lambda shape: pl.BlockSpec(shape, lambda i: (i, 0))
    specs = [row((tile, H)), row((tile, H)), row((tile, H)),
             row((tile, c0.shape[1])), row((tile, c1.shape[1])),
             _full((H, H)), _full((H, H)), _full((1, H))]
    args = [x, s0, s1, c0, c1, nw[0][:H], nw[0][H:], nb[0].reshape(1, H)]
    for w, b in zip(nw[1:], nb[1:]):
        specs += [_full((H, H)), _full((1, H))]
        args += [w, b.reshape(1, H)]
    specs += [_full((1, H)), _full((1, H))]
    args += [ng.reshape(1, H), nbt.reshape(1, H)]
    return pl.pallas_call(
        _node_mlp_body,
        grid=(grid,),
        in_specs=specs,
        out_specs=row((tile, H)),
        out_shape=jax.ShapeDtypeStruct((N, H), jnp.float32),
    )(*args)


def kernel(x, edge_index, edge_attr, ew0, eb0, ew1, eb1, ew2, eb2, ew3, eb3,
           eg, ebt, nw0, nb0, nw1, nb1, nw2, nb2, nw3, nb3, ng, nbt):
    N, H = x.shape
    E = edge_attr.shape[0]
    col = edge_index[1]

    edge_attr_out = _edge_mlp(edge_attr, [ew0, ew1, ew2, ew3],
                              [eb0, eb1, eb2, eb3], eg, ebt, tile=1280)

    s0 = jax.ops.segment_sum(edge_attr_out, col, num_segments=N)
    cnt = jax.ops.segment_sum(jnp.ones((E,), jnp.float32), col, num_segments=N)
    s1 = jnp.zeros_like(s0)
    c0 = jnp.broadcast_to(cnt[:, None], (N, 16))
    c1 = jnp.zeros_like(c0)

    x_out = _node_mlp(x, s0, s1, c0, c1, [nw0, nw1, nw2, nw3],
                      [nb0, nb1, nb2, nb3], ng, nbt, tile=1000)
    return (x_out, edge_attr_out)


# speedup vs baseline: 2.1348x; 1.8687x over previous
import functools

import jax
import jax.numpy as jnp
from jax import lax
from jax.experimental import pallas as pl
from jax.experimental.pallas import tpu as pltpu
from jax.experimental.pallas import tpu_sc as plsc

EPS = 1e-5
NC = 2
NS = 16


def _sc_scatter(edge_rows, col, N):
    E, H = edge_rows.shape
    W = 2 * H
    per_tile = E // NS
    chunk = 80
    iters = per_tile // chunk
    rows_per_tile = (N + NC * NS * 8 - 1) // (NC * NS * 8) * 8
    half = rows_per_tile * NS
    Np = half * NC
    stage_iters = rows_per_tile // chunk
    mesh = plsc.VectorSubcoreMesh(core_axis_name="c", subcore_axis_name="s")

    @functools.partial(
        pl.kernel,
        out_type=jax.ShapeDtypeStruct((Np, 2, H), jnp.float32),
        mesh=mesh,
        scratch_types=[
            pltpu.VMEM_SHARED((half + 8, 2, H), jnp.float32),
            pltpu.VMEM((chunk,), jnp.int32),
            pltpu.VMEM((chunk, 2, H), jnp.float32),
        ],
    )
    def k(rows_hbm, col_hbm, sums_hbm, acc, idx_v, rows_v):
        c = lax.axis_index("c")
        s = lax.axis_index("s")
        base = s * per_tile
        my_rows = s * rows_per_tile
        lo = c * half

        @pl.loop(0, chunk)
        def fill(i):
            for sl in range(2):
                for j in range(H // 16):
                    rows_v[i, sl, pl.ds(16 * j, 16)] = (
                        jnp.zeros((16,), jnp.float32))

        @pl.loop(0, stage_iters)
        def zero(i):
            off = my_rows + i * chunk
            pltpu.sync_copy(rows_v, acc.at[pl.ds(off, chunk)])

        @pl.when(s == 0)
        def zero_trash():
            pltpu.sync_copy(rows_v.at[pl.ds(0, 8)], acc.at[pl.ds(half, 8)])

        one_hot = jnp.where(lax.iota(jnp.int32, 16) == 0,
                            jnp.float32(1), jnp.float32(0))

        @pl.loop(0, chunk)
        def fill1(i):
            rows_v[i, 1, pl.ds(0, 16)] = one_hot

        plsc.subcore_barrier()

        lo16 = jnp.full((16,), lo, jnp.int32)

        @pl.loop(0, iters)
        def body(j):
            off = base + j * chunk
            pltpu.sync_copy(col_hbm.at[pl.ds(off, chunk)], idx_v)
            pltpu.sync_copy(rows_hbm.at[pl.ds(off, chunk)],
                            rows_v.at[:, 0])
            for g in range(chunk // 16):
                v = idx_v[pl.ds(16 * g, 16)] - lo16
                ok = (v >= 0) & (v < half)
                idx_v[pl.ds(16 * g, 16)] = jnp.where(
                    ok, v, jnp.full((16,), half, jnp.int32))
            pltpu.sync_copy(rows_v, acc.at[idx_v], add=True)

        plsc.subcore_barrier()

        @pl.loop(0, stage_iters)
        def pub(i):
            off = my_rows + i * chunk
            pltpu.sync_copy(acc.at[pl.ds(off, chunk)], rows_v)
            pltpu.sync_copy(rows_v, sums_hbm.at[pl.ds(lo + off, chunk)])

    return k(edge_rows, col)


def _edge_mlp_body(a_ref, w0, b0, w1, b1, w2, b2, w3, b3, g, bt, out_ref):
    a = a_ref[...]
    h = jnp.maximum(jnp.dot(a, w0[...], preferred_element_type=jnp.float32) + b0[...], 0.0)
    h = jnp.maximum(jnp.dot(h, w1[...], preferred_element_type=jnp.float32) + b1[...], 0.0)
    h = jnp.maximum(jnp.dot(h, w2[...], preferred_element_type=jnp.float32) + b2[...], 0.0)
    h = jnp.dot(h, w3[...], preferred_element_type=jnp.float32) + b3[...]
    mu = jnp.mean(h, axis=-1, keepdims=True)
    var = jnp.mean((h - mu) ** 2, axis=-1, keepdims=True)
    ln = (h - mu) * jax.lax.rsqrt(var + EPS) * g[...] + bt[...]
    out_ref[...] = a + ln


def _node_mlp_body(x_ref, s_ref, w0a, w0b, b0, w1, b1, w2, b2, w3, b3,
                   g, bt, out_ref):
    x = x_ref[...]
    s = s_ref[...]
    cnt = jnp.maximum(s[:, 1, 0:1], 1.0)
    agg = s[:, 0, :] / cnt
    h = jnp.dot(x, w0a[...], preferred_element_type=jnp.float32)
    h = h + jnp.dot(agg, w0b[...], preferred_element_type=jnp.float32)
    h = jnp.maximum(h + b0[...], 0.0)
    h = jnp.maximum(jnp.dot(h, w1[...], preferred_element_type=jnp.float32) + b1[...], 0.0)
    h = jnp.maximum(jnp.dot(h, w2[...], preferred_element_type=jnp.float32) + b2[...], 0.0)
    h = jnp.dot(h, w3[...], preferred_element_type=jnp.float32) + b3[...]
    mu = jnp.mean(h, axis=-1, keepdims=True)
    var = jnp.mean((h - mu) ** 2, axis=-1, keepdims=True)
    ln = (h - mu) * jax.lax.rsqrt(var + EPS) * g[...] + bt[...]
    out_ref[...] = x + ln


def _full(shape):
    return pl.BlockSpec(shape, lambda i: (0,) * len(shape))


def _edge_mlp(edge_attr, ew, eb, eg, ebt, tile):
    E, H = edge_attr.shape
    grid = E // tile
    specs = [pl.BlockSpec((tile, H), lambda i: (i, 0))]
    args = [edge_attr]
    for w, b in zip(ew, eb):
        specs += [_full((H, H)), _full((1, H))]
        args += [w, b.reshape(1, H)]
    specs += [_full((1, H)), _full((1, H))]
    args += [eg.reshape(1, H), ebt.reshape(1, H)]
    return pl.pallas_call(
        _edge_mlp_body,
        grid=(grid,),
        in_specs=specs,
        out_specs=pl.BlockSpec((tile, H), lambda i: (i, 0)),
        out_shape=jax.ShapeDtypeStruct((E, H), jnp.float32),
    )(*args)


def _node_mlp(x, sums, nw, nb, ng, nbt, tile):
    N, H = x.shape
    grid = N // tile
    row = lambda shape: pl.BlockSpec(shape, lambda i: (i, 0))
    specs = [row((tile, H)),
             pl.BlockSpec((tile, 2, H), lambda i: (i, 0, 0)),
             _full((H, H)), _full((H, H)), _full((1, H))]
    args = [x, sums, nw[0][:H], nw[0][H:], nb[0].reshape(1, H)]
    for w, b in zip(nw[1:], nb[1:]):
        specs += [_full((H, H)), _full((1, H))]
        args += [w, b.reshape(1, H)]
    specs += [_full((1, H)), _full((1, H))]
    args += [ng.reshape(1, H), nbt.reshape(1, H)]
    return pl.pallas_call(
        _node_mlp_body,
        grid=(grid,),
        in_specs=specs,
        out_specs=row((tile, H)),
        out_shape=jax.ShapeDtypeStruct((N, H), jnp.float32),
    )(*args)


def kernel(x, edge_index, edge_attr, ew0, eb0, ew1, eb1, ew2, eb2, ew3, eb3,
           eg, ebt, nw0, nb0, nw1, nb1, nw2, nb2, nw3, nb3, ng, nbt):
    N, H = x.shape
    E = edge_attr.shape[0]
    col = edge_index[1]

    edge_attr_out = _edge_mlp(edge_attr, [ew0, ew1, ew2, ew3],
                              [eb0, eb1, eb2, eb3], eg, ebt, tile=1280)

    sums = _sc_scatter(edge_attr_out, col, N)

    x_out = _node_mlp(x, sums, [nw0, nw1, nw2, nw3],
                      [nb0, nb1, nb2, nb3], ng, nbt, tile=1000)
    return (x_out, edge_attr_out)


# speedup vs baseline: 2.4737x; 1.1588x over previous
import functools

import jax
import jax.numpy as jnp
from jax import lax
from jax.experimental import pallas as pl
from jax.experimental.pallas import tpu as pltpu
from jax.experimental.pallas import tpu_sc as plsc

EPS = 1e-5
NC = 2
NS = 16


def _sc_scatter(edge_rows, col, N):
    E, H = edge_rows.shape
    W = 2 * H
    per_tile = E // NS
    chunk = 160
    iters = per_tile // chunk
    rows_per_tile = (N + NC * NS * 8 - 1) // (NC * NS * 8) * 8
    half = rows_per_tile * NS
    Np = half * NC
    stage_iters = rows_per_tile // chunk
    mesh = plsc.VectorSubcoreMesh(core_axis_name="c", subcore_axis_name="s")

    @functools.partial(
        pl.kernel,
        out_type=jax.ShapeDtypeStruct((Np, 2, H), jnp.float32),
        mesh=mesh,
        scratch_types=[
            pltpu.VMEM_SHARED((half + 8, 2, H), jnp.float32),
            pltpu.VMEM((chunk,), jnp.int32),
            pltpu.VMEM((chunk, 2, H), jnp.float32),
        ],
    )
    def k(rows_hbm, col_hbm, sums_hbm, acc, idx_v, rows_v):
        c = lax.axis_index("c")
        s = lax.axis_index("s")
        base = s * per_tile
        my_rows = s * rows_per_tile
        lo = c * half

        @pl.loop(0, chunk)
        def fill(i):
            for sl in range(2):
                for j in range(H // 16):
                    rows_v[i, sl, pl.ds(16 * j, 16)] = (
                        jnp.zeros((16,), jnp.float32))

        @pl.loop(0, stage_iters)
        def zero(i):
            off = my_rows + i * chunk
            pltpu.sync_copy(rows_v, acc.at[pl.ds(off, chunk)])

        @pl.when(s == 0)
        def zero_trash():
            pltpu.sync_copy(rows_v.at[pl.ds(0, 8)], acc.at[pl.ds(half, 8)])

        one_hot = jnp.where(lax.iota(jnp.int32, 16) == 0,
                            jnp.float32(1), jnp.float32(0))

        @pl.loop(0, chunk)
        def fill1(i):
            rows_v[i, 1, pl.ds(0, 16)] = one_hot

        plsc.subcore_barrier()

        lo16 = jnp.full((16,), lo, jnp.int32)

        @pl.loop(0, iters)
        def body(j):
            off = base + j * chunk
            pltpu.sync_copy(col_hbm.at[pl.ds(off, chunk)], idx_v)
            pltpu.sync_copy(rows_hbm.at[pl.ds(off, chunk)],
                            rows_v.at[:, 0])
            for g in range(chunk // 16):
                v = idx_v[pl.ds(16 * g, 16)] - lo16
                ok = (v >= 0) & (v < half)
                idx_v[pl.ds(16 * g, 16)] = jnp.where(
                    ok, v, jnp.full((16,), half, jnp.int32))
            pltpu.sync_copy(rows_v, acc.at[idx_v], add=True)

        plsc.subcore_barrier()

        @pl.loop(0, stage_iters)
        def pub(i):
            off = my_rows + i * chunk
            pltpu.sync_copy(acc.at[pl.ds(off, chunk)], rows_v)
            pltpu.sync_copy(rows_v, sums_hbm.at[pl.ds(lo + off, chunk)])

    return k(edge_rows, col)


def _edge_mlp_body(a_ref, w0, b0, w1, b1, w2, b2, w3, b3, g, bt, out_ref):
    a = a_ref[...]
    h = jnp.maximum(jnp.dot(a, w0[...], preferred_element_type=jnp.float32) + b0[...], 0.0)
    h = jnp.maximum(jnp.dot(h, w1[...], preferred_element_type=jnp.float32) + b1[...], 0.0)
    h = jnp.maximum(jnp.dot(h, w2[...], preferred_element_type=jnp.float32) + b2[...], 0.0)
    h = jnp.dot(h, w3[...], preferred_element_type=jnp.float32) + b3[...]
    mu = jnp.mean(h, axis=-1, keepdims=True)
    var = jnp.mean((h - mu) ** 2, axis=-1, keepdims=True)
    ln = (h - mu) * jax.lax.rsqrt(var + EPS) * g[...] + bt[...]
    out_ref[...] = a + ln


def _node_mlp_body(x_ref, s_ref, w0a, w0b, b0, w1, b1, w2, b2, w3, b3,
                   g, bt, out_ref):
    x = x_ref[...]
    s = s_ref[...]
    cnt = jnp.maximum(s[:, 1, 0:1], 1.0)
    agg = s[:, 0, :] / cnt
    h = jnp.dot(x, w0a[...], preferred_element_type=jnp.float32)
    h = h + jnp.dot(agg, w0b[...], preferred_element_type=jnp.float32)
    h = jnp.maximum(h + b0[...], 0.0)
    h = jnp.maximum(jnp.dot(h, w1[...], preferred_element_type=jnp.float32) + b1[...], 0.0)
    h = jnp.maximum(jnp.dot(h, w2[...], preferred_element_type=jnp.float32) + b2[...], 0.0)
    h = jnp.dot(h, w3[...], preferred_element_type=jnp.float32) + b3[...]
    mu = jnp.mean(h, axis=-1, keepdims=True)
    var = jnp.mean((h - mu) ** 2, axis=-1, keepdims=True)
    ln = (h - mu) * jax.lax.rsqrt(var + EPS) * g[...] + bt[...]
    out_ref[...] = x + ln


def _full(shape):
    return pl.BlockSpec(shape, lambda i: (0,) * len(shape))


def _edge_mlp(edge_attr, ew, eb, eg, ebt, tile):
    E, H = edge_attr.shape
    grid = E // tile
    specs = [pl.BlockSpec((tile, H), lambda i: (i, 0))]
    args = [edge_attr]
    for w, b in zip(ew, eb):
        specs += [_full((H, H)), _full((1, H))]
        args += [w, b.reshape(1, H)]
    specs += [_full((1, H)), _full((1, H))]
    args += [eg.reshape(1, H), ebt.reshape(1, H)]
    return pl.pallas_call(
        _edge_mlp_body,
        grid=(grid,),
        in_specs=specs,
        out_specs=pl.BlockSpec((tile, H), lambda i: (i, 0)),
        out_shape=jax.ShapeDtypeStruct((E, H), jnp.float32),
    )(*args)


def _node_mlp(x, sums, nw, nb, ng, nbt, tile):
    N, H = x.shape
    grid = N // tile
    row = lambda shape: pl.BlockSpec(shape, lambda i: (i, 0))
    specs = [row((tile, H)),
             pl.BlockSpec((tile, 2, H), lambda i: (i, 0, 0)),
             _full((H, H)), _full((H, H)), _full((1, H))]
    args = [x, sums, nw[0][:H], nw[0][H:], nb[0].reshape(1, H)]
    for w, b in zip(nw[1:], nb[1:]):
        specs += [_full((H, H)), _full((1, H))]
        args += [w, b.reshape(1, H)]
    specs += [_full((1, H)), _full((1, H))]
    args += [ng.reshape(1, H), nbt.reshape(1, H)]
    return pl.pallas_call(
        _node_mlp_body,
        grid=(grid,),
        in_specs=specs,
        out_specs=row((tile, H)),
        out_shape=jax.ShapeDtypeStruct((N, H), jnp.float32),
    )(*args)


def kernel(x, edge_index, edge_attr, ew0, eb0, ew1, eb1, ew2, eb2, ew3, eb3,
           eg, ebt, nw0, nb0, nw1, nb1, nw2, nb2, nw3, nb3, ng, nbt):
    N, H = x.shape
    E = edge_attr.shape[0]
    col = edge_index[1]

    edge_attr_out = _edge_mlp(edge_attr, [ew0, ew1, ew2, ew3],
                              [eb0, eb1, eb2, eb3], eg, ebt, tile=1280)

    sums = _sc_scatter(edge_attr_out, col, N)

    x_out = _node_mlp(x, sums, [nw0, nw1, nw2, nw3],
                      [nb0, nb1, nb2, nb3], ng, nbt, tile=1000)
    return (x_out, edge_attr_out)


# speedup vs baseline: 2.9233x; 1.1817x over previous
import functools

import jax
import jax.numpy as jnp
from jax import lax
from jax.experimental import pallas as pl
from jax.experimental.pallas import tpu as pltpu
from jax.experimental.pallas import tpu_sc as plsc

EPS = 1e-5
NC = 2
NS = 16


def _sc_scatter(edge_rows, col, N):
    E, H = edge_rows.shape
    W = 2 * H
    per_tile = E // NS
    chunk = 80
    iters = per_tile // chunk
    rows_per_tile = (N + NC * NS * 8 - 1) // (NC * NS * 8) * 8
    half = rows_per_tile * NS
    Np = half * NC
    stage_iters = rows_per_tile // chunk
    mesh = plsc.VectorSubcoreMesh(core_axis_name="c", subcore_axis_name="s")

    @functools.partial(
        pl.kernel,
        out_type=jax.ShapeDtypeStruct((Np, 2, H), jnp.float32),
        mesh=mesh,
        scratch_types=[
            pltpu.VMEM_SHARED((half + 8, 2, H), jnp.float32),
            pltpu.VMEM((chunk,), jnp.int32),
            pltpu.VMEM((chunk, 2, H), jnp.float32),
            pltpu.VMEM((chunk,), jnp.int32),
            pltpu.VMEM((chunk, 2, H), jnp.float32),
            pltpu.SemaphoreType.DMA,
            pltpu.SemaphoreType.DMA,
        ],
    )
    def k(rows_hbm, col_hbm, sums_hbm, acc, idx_v0, rows_v0,
          idx_v1, rows_v1, rsem0, rsem1):
        idx_v, rows_v = idx_v0, rows_v0
        c = lax.axis_index("c")
        s = lax.axis_index("s")
        base = s * per_tile
        my_rows = s * rows_per_tile
        lo = c * half

        @pl.loop(0, chunk)
        def fill(i):
            for sl in range(2):
                for j in range(H // 16):
                    rows_v0[i, sl, pl.ds(16 * j, 16)] = (
                        jnp.zeros((16,), jnp.float32))
                    rows_v1[i, sl, pl.ds(16 * j, 16)] = (
                        jnp.zeros((16,), jnp.float32))

        @pl.loop(0, stage_iters)
        def zero(i):
            off = my_rows + i * chunk
            pltpu.sync_copy(rows_v, acc.at[pl.ds(off, chunk)])

        @pl.when(s == 0)
        def zero_trash():
            pltpu.sync_copy(rows_v.at[pl.ds(0, 8)], acc.at[pl.ds(half, 8)])

        one_hot = jnp.where(lax.iota(jnp.int32, 16) == 0,
                            jnp.float32(1), jnp.float32(0))

        @pl.loop(0, chunk)
        def fill1(i):
            rows_v0[i, 1, pl.ds(0, 16)] = one_hot
            rows_v1[i, 1, pl.ds(0, 16)] = one_hot

        plsc.subcore_barrier()

        lo16 = jnp.full((16,), lo, jnp.int32)
        bufs = ((idx_v0, rows_v0, rsem0), (idx_v1, rows_v1, rsem1))

        def start_read(b, off):
            ib, rb, sem = bufs[b]
            pltpu.async_copy(col_hbm.at[pl.ds(off, chunk)], ib, sem)
            pltpu.async_copy(rows_hbm.at[pl.ds(off, chunk)], rb.at[:, 0], sem)

        def wait_read(b):
            ib, rb, sem = bufs[b]
            pltpu.make_async_copy(col_hbm.at[pl.ds(0, chunk)], ib, sem).wait()
            pltpu.make_async_copy(rows_hbm.at[pl.ds(0, chunk)],
                                  rb.at[:, 0], sem).wait()

        start_read(0, base)
        start_read(1, base + chunk)

        @pl.loop(0, iters, step=2)
        def body(t):
            for b in range(2):
                ib, rb, _ = bufs[b]
                jj = t + b
                wait_read(b)
                for g in range(chunk // 16):
                    v = ib[pl.ds(16 * g, 16)] - lo16
                    ok = (v >= 0) & (v < half)
                    ib[pl.ds(16 * g, 16)] = jnp.where(
                        ok, v, jnp.full((16,), half, jnp.int32))
                pltpu.sync_copy(rb, acc.at[ib], add=True)

                @pl.when(jj + 2 < iters)
                def prefetch():
                    start_read(b, base + (jj + 2) * chunk)

        plsc.subcore_barrier()

        @pl.loop(0, stage_iters)
        def pub(i):
            off = my_rows + i * chunk
            pltpu.sync_copy(acc.at[pl.ds(off, chunk)], rows_v)
            pltpu.sync_copy(rows_v, sums_hbm.at[pl.ds(lo + off, chunk)])

    return k(edge_rows, col)


def _edge_mlp_body(a_ref, w0, b0, w1, b1, w2, b2, w3, b3, g, bt, out_ref):
    a = a_ref[...]
    h = jnp.maximum(jnp.dot(a, w0[...], preferred_element_type=jnp.float32) + b0[...], 0.0)
    h = jnp.maximum(jnp.dot(h, w1[...], preferred_element_type=jnp.float32) + b1[...], 0.0)
    h = jnp.maximum(jnp.dot(h, w2[...], preferred_element_type=jnp.float32) + b2[...], 0.0)
    h = jnp.dot(h, w3[...], preferred_element_type=jnp.float32) + b3[...]
    mu = jnp.mean(h, axis=-1, keepdims=True)
    var = jnp.mean((h - mu) ** 2, axis=-1, keepdims=True)
    ln = (h - mu) * jax.lax.rsqrt(var + EPS) * g[...] + bt[...]
    out_ref[...] = a + ln


def _node_mlp_body(x_ref, s_ref, w0a, w0b, b0, w1, b1, w2, b2, w3, b3,
                   g, bt, out_ref):
    x = x_ref[...]
    s = s_ref[...]
    cnt = jnp.maximum(s[:, 1, 0:1], 1.0)
    agg = s[:, 0, :] / cnt
    h = jnp.dot(x, w0a[...], preferred_element_type=jnp.float32)
    h = h + jnp.dot(agg, w0b[...], preferred_element_type=jnp.float32)
    h = jnp.maximum(h + b0[...], 0.0)
    h = jnp.maximum(jnp.dot(h, w1[...], preferred_element_type=jnp.float32) + b1[...], 0.0)
    h = jnp.maximum(jnp.dot(h, w2[...], preferred_element_type=jnp.float32) + b2[...], 0.0)
    h = jnp.dot(h, w3[...], preferred_element_type=jnp.float32) + b3[...]
    mu = jnp.mean(h, axis=-1, keepdims=True)
    var = jnp.mean((h - mu) ** 2, axis=-1, keepdims=True)
    ln = (h - mu) * jax.lax.rsqrt(var + EPS) * g[...] + bt[...]
    out_ref[...] = x + ln


def _full(shape):
    return pl.BlockSpec(shape, lambda i: (0,) * len(shape))


def _edge_mlp(edge_attr, ew, eb, eg, ebt, tile):
    E, H = edge_attr.shape
    grid = E // tile
    specs = [pl.BlockSpec((tile, H), lambda i: (i, 0))]
    args = [edge_attr]
    for w, b in zip(ew, eb):
        specs += [_full((H, H)), _full((1, H))]
        args += [w, b.reshape(1, H)]
    specs += [_full((1, H)), _full((1, H))]
    args += [eg.reshape(1, H), ebt.reshape(1, H)]
    return pl.pallas_call(
        _edge_mlp_body,
        grid=(grid,),
        in_specs=specs,
        out_specs=pl.BlockSpec((tile, H), lambda i: (i, 0)),
        out_shape=jax.ShapeDtypeStruct((E, H), jnp.float32),
    )(*args)


def _node_mlp(x, sums, nw, nb, ng, nbt, tile):
    N, H = x.shape
    grid = N // tile
    row = lambda shape: pl.BlockSpec(shape, lambda i: (i, 0))
    specs = [row((tile, H)),
             pl.BlockSpec((tile, 2, H), lambda i: (i, 0, 0)),
             _full((H, H)), _full((H, H)), _full((1, H))]
    args = [x, sums, nw[0][:H], nw[0][H:], nb[0].reshape(1, H)]
    for w, b in zip(nw[1:], nb[1:]):
        specs += [_full((H, H)), _full((1, H))]
        args += [w, b.reshape(1, H)]
    specs += [_full((1, H)), _full((1, H))]
    args += [ng.reshape(1, H), nbt.reshape(1, H)]
    return pl.pallas_call(
        _node_mlp_body,
        grid=(grid,),
        in_specs=specs,
        out_specs=row((tile, H)),
        out_shape=jax.ShapeDtypeStruct((N, H), jnp.float32),
    )(*args)


def kernel(x, edge_index, edge_attr, ew0, eb0, ew1, eb1, ew2, eb2, ew3, eb3,
           eg, ebt, nw0, nb0, nw1, nb1, nw2, nb2, nw3, nb3, ng, nbt):
    N, H = x.shape
    E = edge_attr.shape[0]
    col = edge_index[1]

    edge_attr_out = _edge_mlp(edge_attr, [ew0, ew1, ew2, ew3],
                              [eb0, eb1, eb2, eb3], eg, ebt, tile=1280)

    sums = _sc_scatter(edge_attr_out, col, N)

    x_out = _node_mlp(x, sums, [nw0, nw1, nw2, nw3],
                      [nb0, nb1, nb2, nb3], ng, nbt, tile=1000)
    return (x_out, edge_attr_out)
